# baseline (speedup 1.0000x reference)
import functools

import jax
import jax.numpy as jnp
from jax import lax
from jax.experimental import pallas as pl
from jax.experimental.pallas import tpu as pltpu
from jax.experimental.pallas import tpu_sc as plsc

N = 10000
D = 128
NC = 2
NS = 16
NW = NC * NS
K = 128
ROWS_PER_TILE = 632
N_PAD = NS * ROWS_PER_TILE
DUMMY_ROW = N


def _pub_sizes(k):
    sizes = [k] * (ROWS_PER_TILE // k)
    if ROWS_PER_TILE % k:
        sizes.append(ROWS_PER_TILE % k)
    return sizes


def _sc_agg_body(ept, h_hbm, src_hbm, dst_hbm, out_hbm,
                 sbuf, dbuf, rows, isem, dsem, gsem, acc):
    c = lax.axis_index("c")
    s = lax.axis_index("s")
    wid = c * NS + s
    ch_per_tile = ept // K

    zv = jnp.zeros((16,), jnp.float32)

    @pl.loop(0, K)
    def _(i):
        for j in range(D // 16):
            rows[i, pl.ds(j * 16, 16)] = zv

    r0 = s * ROWS_PER_TILE
    for sz in _pub_sizes(K):
        pltpu.sync_copy(rows.at[pl.ds(0, sz)], acc.at[pl.ds(r0, sz)])
        r0 += sz

    plsc.subcore_barrier()

    @pl.loop(0, ch_per_tile)
    def _(ch):
        off = pl.multiple_of(wid * ept + ch * K, K)
        pltpu.async_copy(src_hbm.at[pl.ds(off, K)], sbuf, isem).wait()
        pltpu.async_copy(dst_hbm.at[pl.ds(off, K)], dbuf, dsem).wait()
        pltpu.async_copy(h_hbm.at[sbuf], rows, gsem).wait()
        pltpu.sync_copy(rows, acc.at[dbuf], add=True)

    plsc.subcore_barrier()

    r0 = s * ROWS_PER_TILE
    for sz in _pub_sizes(K):
        pltpu.sync_copy(acc.at[pl.ds(r0, sz)], rows.at[pl.ds(0, sz)])
        pltpu.sync_copy(rows.at[pl.ds(0, sz)], out_hbm.at[c, pl.ds(r0, sz)])
        r0 += sz


def _sc_deg_body(ept, dst_hbm, deg_hbm, dbuf, ones16, dsem, degacc):
    c = lax.axis_index("c")
    s = lax.axis_index("s")
    wid = c * NS + s
    ch_per_tile = ept // K

    zv = jnp.zeros((16,), jnp.float32)

    @pl.loop(0, K)
    def _(i):
        ones16[i, pl.ds(0, 16)] = zv

    r0 = s * ROWS_PER_TILE
    for sz in _pub_sizes(K):
        pltpu.sync_copy(ones16.at[pl.ds(0, sz)], degacc.at[pl.ds(r0, sz)])
        r0 += sz

    ov = jnp.ones((16,), jnp.float32)

    @pl.loop(0, K)
    def _(i):
        ones16[i, pl.ds(0, 16)] = ov

    plsc.subcore_barrier()

    @pl.loop(0, ch_per_tile)
    def _(ch):
        off = pl.multiple_of(wid * ept + ch * K, K)
        pltpu.async_copy(dst_hbm.at[pl.ds(off, K)], dbuf, dsem).wait()
        pltpu.sync_copy(ones16, degacc.at[dbuf], add=True)

    plsc.subcore_barrier()

    r0 = s * ROWS_PER_TILE
    for sz in _pub_sizes(K):
        pltpu.sync_copy(degacc.at[pl.ds(r0, sz)], ones16.at[pl.ds(0, sz)])
        pltpu.sync_copy(ones16.at[pl.ds(0, sz)], deg_hbm.at[c, pl.ds(r0, sz)])
        r0 += sz


def _make_sc_agg(ept):
    mesh = plsc.VectorSubcoreMesh(core_axis_name="c", subcore_axis_name="s")
    return pl.kernel(
        functools.partial(_sc_agg_body, ept),
        out_type=jax.ShapeDtypeStruct((NC, N_PAD, D), jnp.float32),
        mesh=mesh,
        scratch_types=(
            pltpu.VMEM((K,), jnp.int32),
            pltpu.VMEM((K,), jnp.int32),
            pltpu.VMEM((K, D), jnp.float32),
            pltpu.SemaphoreType.DMA,
            pltpu.SemaphoreType.DMA,
            pltpu.SemaphoreType.DMA,
            pltpu.VMEM_SHARED((N_PAD, D), jnp.float32),
        ),
    )


def _make_sc_deg(ept):
    mesh = plsc.VectorSubcoreMesh(core_axis_name="c", subcore_axis_name="s")
    return pl.kernel(
        functools.partial(_sc_deg_body, ept),
        out_type=jax.ShapeDtypeStruct((NC, N_PAD, 16), jnp.float32),
        mesh=mesh,
        scratch_types=(
            pltpu.VMEM((K,), jnp.int32),
            pltpu.VMEM((K, 16), jnp.float32),
            pltpu.SemaphoreType.DMA,
            pltpu.VMEM_SHARED((N_PAD, 16), jnp.float32),
        ),
    )


def _tc_mean_layer_body(p0, p1, d0, d1, h, wl, bl, wr, out):
    deg = d0[0, :, 0:1] + d1[0, :, 0:1]
    inv = 1.0 / jnp.maximum(deg, 1.0)
    agg = (p0[0] + p1[0]) * inv
    y = (jnp.dot(agg, wl[...], preferred_element_type=jnp.float32)
         + bl[...]
         + jnp.dot(h[...], wr[...], preferred_element_type=jnp.float32))
    out[...] = jnp.maximum(y, 0.0)


def _tc_final_layer_body(p0, p1, h, wl, bl, wr, wlin, blin, out):
    agg = p0[0] + p1[0]
    y = (jnp.dot(agg, wl[...], preferred_element_type=jnp.float32)
         + bl[...]
         + jnp.dot(h[...], wr[...], preferred_element_type=jnp.float32))
    hh = jnp.maximum(y, 0.0)
    o = jnp.dot(hh, wlin[...], preferred_element_type=jnp.float32) + blin[...]
    e = jnp.exp(o - jnp.max(o, axis=1, keepdims=True))
    out[...] = e / jnp.sum(e, axis=1, keepdims=True)


_BM = 1264


def _part_spec(width, part):
    return pl.BlockSpec((1, _BM, width), lambda i, _p=part: (_p, i, 0))


def _row_spec(width):
    return pl.BlockSpec((_BM, width), lambda i: (i, 0))


def _full_spec(r, ccol):
    return pl.BlockSpec((r, ccol), lambda i: (0, 0))


def _tc_mean_layer(p, dp, h, wl, bl, wr):
    return pl.pallas_call(
        _tc_mean_layer_body,
        grid=(N_PAD // _BM,),
        in_specs=[
            _part_spec(D, 0), _part_spec(D, 1),
            _part_spec(16, 0), _part_spec(16, 1),
            _row_spec(D), _full_spec(D, D), _full_spec(1, D), _full_spec(D, D),
        ],
        out_specs=_row_spec(D),
        out_shape=jax.ShapeDtypeStruct((N_PAD, D), jnp.float32),
    )(p, p, dp, dp, h, wl, bl.reshape(1, D), wr)


def _tc_final_layer(p, h, wl, bl, wr, wlin, blin):
    return pl.pallas_call(
        _tc_final_layer_body,
        grid=(N_PAD // _BM,),
        in_specs=[
            _part_spec(D, 0), _part_spec(D, 1), _row_spec(D),
            _full_spec(D, D), _full_spec(1, D), _full_spec(D, D),
            _full_spec(D, 1), _full_spec(1, 1),
        ],
        out_specs=_row_spec(1),
        out_shape=jax.ShapeDtypeStruct((N_PAD, 1), jnp.float32),
    )(p, p, h, wl, bl.reshape(1, D), wr, wlin, blin.reshape(1, 1))


def kernel(x, edge_index, W1l, b1l, W1r, W2l, b2l, W2r, W3l, b3l, W3r,
           Wlin, blin):
    e = edge_index.shape[1]
    ept = -(-e // (NW * K)) * K
    e_pad = ept * NW
    src = edge_index[0].astype(jnp.int32)
    dst = edge_index[1].astype(jnp.int32)
    pad = e_pad - e
    if pad:
        fill = jnp.arange(pad, dtype=jnp.int32)
        src = jnp.concatenate([src, fill % N])
        dst = jnp.concatenate([dst, DUMMY_ROW + fill % (N_PAD - N)])
    xp = jnp.concatenate([x, jnp.zeros((N_PAD - N, D), x.dtype)])

    sc_agg = _make_sc_agg(ept)
    sc_deg = _make_sc_deg(ept)

    dp = sc_deg(dst)
    p = sc_agg(xp, src, dst)
    h1 = _tc_mean_layer(p, dp, xp, W1l, b1l, W1r)
    p = sc_agg(h1, src, dst)
    h2 = _tc_mean_layer(p, dp, h1, W2l, b2l, W2r)
    p = sc_agg(h2, src, dst)
    out = _tc_final_layer(p, h2, W3l, b3l, W3r, Wlin, blin)
    return out[:N]


# speedup vs baseline: 1.3623x; 1.3623x over previous
import functools

import jax
import jax.numpy as jnp
from jax import lax
from jax.experimental import pallas as pl
from jax.experimental.pallas import tpu as pltpu
from jax.experimental.pallas import tpu_sc as plsc

N = 10000
D = 128
NC = 2
NS = 16
NW = NC * NS
K = 128
NBUF = 2
ROWS_PER_TILE = 632
N_PAD = NS * ROWS_PER_TILE
DUMMY_ROW = N


def _pub_sizes(k):
    sizes = [k] * (ROWS_PER_TILE // k)
    if ROWS_PER_TILE % k:
        sizes.append(ROWS_PER_TILE % k)
    return sizes


def _sc_agg_body(ept, h_hbm, src_hbm, dst_hbm, out_hbm,
                 sbuf, dbuf, rows, isem, dsem, gsem, acc):
    c = lax.axis_index("c")
    s = lax.axis_index("s")
    wid = c * NS + s
    ch_per_tile = ept // K

    zv = jnp.zeros((16,), jnp.float32)

    @pl.loop(0, K)
    def _(i):
        for j in range(D // 16):
            rows[0][i, pl.ds(j * 16, 16)] = zv

    r0 = s * ROWS_PER_TILE
    for sz in _pub_sizes(K):
        pltpu.sync_copy(rows[0].at[pl.ds(0, sz)], acc.at[pl.ds(r0, sz)])
        r0 += sz

    plsc.subcore_barrier()

    @pl.loop(0, ch_per_tile, step=NBUF)
    def _(i):
        icps = []
        for b in range(NBUF):
            off = pl.multiple_of(wid * ept + (i + b) * K, K)
            icps.append((
                pltpu.async_copy(src_hbm.at[pl.ds(off, K)], sbuf[b], isem[b]),
                pltpu.async_copy(dst_hbm.at[pl.ds(off, K)], dbuf[b], dsem[b]),
            ))
        gcps = []
        for b in range(NBUF):
            icps[b][0].wait()
            gcps.append(pltpu.async_copy(h_hbm.at[sbuf[b]], rows[b], gsem[b]))
        for b in range(NBUF):
            gcps[b].wait()
            icps[b][1].wait()
            pltpu.sync_copy(rows[b], acc.at[dbuf[b]], add=True)

    plsc.subcore_barrier()

    r0 = s * ROWS_PER_TILE
    for sz in _pub_sizes(K):
        pltpu.sync_copy(acc.at[pl.ds(r0, sz)], rows[0].at[pl.ds(0, sz)])
        pltpu.sync_copy(rows[0].at[pl.ds(0, sz)], out_hbm.at[c, pl.ds(r0, sz)])
        r0 += sz


def _sc_deg_body(ept, dst_hbm, deg_hbm, dbuf, ones16, dsem, degacc):
    c = lax.axis_index("c")
    s = lax.axis_index("s")
    wid = c * NS + s
    ch_per_tile = ept // K

    zv = jnp.zeros((16,), jnp.float32)

    @pl.loop(0, K)
    def _(i):
        ones16[i, pl.ds(0, 16)] = zv

    r0 = s * ROWS_PER_TILE
    for sz in _pub_sizes(K):
        pltpu.sync_copy(ones16.at[pl.ds(0, sz)], degacc.at[pl.ds(r0, sz)])
        r0 += sz

    ov = jnp.ones((16,), jnp.float32)

    @pl.loop(0, K)
    def _(i):
        ones16[i, pl.ds(0, 16)] = ov

    plsc.subcore_barrier()

    @pl.loop(0, ch_per_tile)
    def _(ch):
        off = pl.multiple_of(wid * ept + ch * K, K)
        pltpu.async_copy(dst_hbm.at[pl.ds(off, K)], dbuf, dsem).wait()
        pltpu.sync_copy(ones16, degacc.at[dbuf], add=True)

    plsc.subcore_barrier()

    r0 = s * ROWS_PER_TILE
    for sz in _pub_sizes(K):
        pltpu.sync_copy(degacc.at[pl.ds(r0, sz)], ones16.at[pl.ds(0, sz)])
        pltpu.sync_copy(ones16.at[pl.ds(0, sz)], deg_hbm.at[c, pl.ds(r0, sz)])
        r0 += sz


def _make_sc_agg(ept):
    mesh = plsc.VectorSubcoreMesh(core_axis_name="c", subcore_axis_name="s")
    return pl.kernel(
        functools.partial(_sc_agg_body, ept),
        out_type=jax.ShapeDtypeStruct((NC, N_PAD, D), jnp.float32),
        mesh=mesh,
        scratch_types=(
            tuple(pltpu.VMEM((K,), jnp.int32) for _ in range(NBUF)),
            tuple(pltpu.VMEM((K,), jnp.int32) for _ in range(NBUF)),
            tuple(pltpu.VMEM((K, D), jnp.float32) for _ in range(NBUF)),
            tuple(pltpu.SemaphoreType.DMA for _ in range(NBUF)),
            tuple(pltpu.SemaphoreType.DMA for _ in range(NBUF)),
            tuple(pltpu.SemaphoreType.DMA for _ in range(NBUF)),
            pltpu.VMEM_SHARED((N_PAD, D), jnp.float32),
        ),
    )


def _make_sc_deg(ept):
    mesh = plsc.VectorSubcoreMesh(core_axis_name="c", subcore_axis_name="s")
    return pl.kernel(
        functools.partial(_sc_deg_body, ept),
        out_type=jax.ShapeDtypeStruct((NC, N_PAD, 16), jnp.float32),
        mesh=mesh,
        scratch_types=(
            pltpu.VMEM((K,), jnp.int32),
            pltpu.VMEM((K, 16), jnp.float32),
            pltpu.SemaphoreType.DMA,
            pltpu.VMEM_SHARED((N_PAD, 16), jnp.float32),
        ),
    )


def _tc_mean_layer_body(p0, p1, d0, d1, h, wl, bl, wr, out):
    deg = d0[0, :, 0:1] + d1[0, :, 0:1]
    inv = 1.0 / jnp.maximum(deg, 1.0)
    agg = (p0[0] + p1[0]) * inv
    y = (jnp.dot(agg, wl[...], preferred_element_type=jnp.float32)
         + bl[...]
         + jnp.dot(h[...], wr[...], preferred_element_type=jnp.float32))
    out[...] = jnp.maximum(y, 0.0)


def _tc_final_layer_body(p0, p1, h, wl, bl, wr, wlin, blin, out):
    agg = p0[0] + p1[0]
    y = (jnp.dot(agg, wl[...], preferred_element_type=jnp.float32)
         + bl[...]
         + jnp.dot(h[...], wr[...], preferred_element_type=jnp.float32))
    hh = jnp.maximum(y, 0.0)
    o = jnp.dot(hh, wlin[...], preferred_element_type=jnp.float32) + blin[...]
    e = jnp.exp(o - jnp.max(o, axis=1, keepdims=True))
    out[...] = e / jnp.sum(e, axis=1, keepdims=True)


_BM = 1264


def _part_spec(width, part):
    return pl.BlockSpec((1, _BM, width), lambda i, _p=part: (_p, i, 0))


def _row_spec(width):
    return pl.BlockSpec((_BM, width), lambda i: (i, 0))


def _full_spec(r, ccol):
    return pl.BlockSpec((r, ccol), lambda i: (0, 0))


def _tc_mean_layer(p, dp, h, wl, bl, wr):
    return pl.pallas_call(
        _tc_mean_layer_body,
        grid=(N_PAD // _BM,),
        in_specs=[
            _part_spec(D, 0), _part_spec(D, 1),
            _part_spec(16, 0), _part_spec(16, 1),
            _row_spec(D), _full_spec(D, D), _full_spec(1, D), _full_spec(D, D),
        ],
        out_specs=_row_spec(D),
        out_shape=jax.ShapeDtypeStruct((N_PAD, D), jnp.float32),
    )(p, p, dp, dp, h, wl, bl.reshape(1, D), wr)


def _tc_final_layer(p, h, wl, bl, wr, wlin, blin):
    return pl.pallas_call(
        _tc_final_layer_body,
        grid=(N_PAD // _BM,),
        in_specs=[
            _part_spec(D, 0), _part_spec(D, 1), _row_spec(D),
            _full_spec(D, D), _full_spec(1, D), _full_spec(D, D),
            _full_spec(D, 1), _full_spec(1, 1),
        ],
        out_specs=_row_spec(1),
        out_shape=jax.ShapeDtypeStruct((N_PAD, 1), jnp.float32),
    )(p, p, h, wl, bl.reshape(1, D), wr, wlin, blin.reshape(1, 1))


def kernel(x, edge_index, W1l, b1l, W1r, W2l, b2l, W2r, W3l, b3l, W3r,
           Wlin, blin):
    e = edge_index.shape[1]
    q = K * NBUF
    ept = -(-e // (NW * q)) * q
    e_pad = ept * NW
    src = edge_index[0].astype(jnp.int32)
    dst = edge_index[1].astype(jnp.int32)
    pad = e_pad - e
    if pad:
        fill = jnp.arange(pad, dtype=jnp.int32)
        src = jnp.concatenate([src, fill % N])
        dst = jnp.concatenate([dst, DUMMY_ROW + fill % (N_PAD - N)])
    xp = jnp.concatenate([x, jnp.zeros((N_PAD - N, D), x.dtype)])

    sc_agg = _make_sc_agg(ept)
    sc_deg = _make_sc_deg(ept)

    dp = sc_deg(dst)
    p = sc_agg(xp, src, dst)
    h1 = _tc_mean_layer(p, dp, xp, W1l, b1l, W1r)
    p = sc_agg(h1, src, dst)
    h2 = _tc_mean_layer(p, dp, h1, W2l, b2l, W2r)
    p = sc_agg(h2, src, dst)
    out = _tc_final_layer(p, h2, W3l, b3l, W3r, Wlin, blin)
    return out[:N]


# speedup vs baseline: 1.4262x; 1.0469x over previous
import functools

import jax
import jax.numpy as jnp
from jax import lax
from jax.experimental import pallas as pl
from jax.experimental.pallas import tpu as pltpu
from jax.experimental.pallas import tpu_sc as plsc

N = 10000
D = 128
NC = 2
NS = 16
NW = NC * NS
K = 128
NBUF = 3
ROWS_PER_TILE = 632
N_PAD = NS * ROWS_PER_TILE
DUMMY_ROW = N


def _pub_sizes(k):
    sizes = [k] * (ROWS_PER_TILE // k)
    if ROWS_PER_TILE % k:
        sizes.append(ROWS_PER_TILE % k)
    return sizes


def _sc_agg_body(ept, h_hbm, src_hbm, dst_hbm, out_hbm,
                 sbuf, dbuf, rows, isem, dsem, gsem, acc):
    c = lax.axis_index("c")
    s = lax.axis_index("s")
    wid = c * NS + s
    ch_per_tile = ept // K

    zv = jnp.zeros((16,), jnp.float32)

    @pl.loop(0, K)
    def _(i):
        for j in range(D // 16):
            rows[0][i, pl.ds(j * 16, 16)] = zv

    r0 = s * ROWS_PER_TILE
    for sz in _pub_sizes(K):
        pltpu.sync_copy(rows[0].at[pl.ds(0, sz)], acc.at[pl.ds(r0, sz)])
        r0 += sz

    plsc.subcore_barrier()

    @pl.loop(0, ch_per_tile, step=NBUF)
    def _(i):
        icps = []
        for b in range(NBUF):
            off = pl.multiple_of(wid * ept + (i + b) * K, K)
            icps.append((
                pltpu.async_copy(src_hbm.at[pl.ds(off, K)], sbuf[b], isem[b]),
                pltpu.async_copy(dst_hbm.at[pl.ds(off, K)], dbuf[b], dsem[b]),
            ))
        gcps = []
        for b in range(NBUF):
            icps[b][0].wait()
            gcps.append(pltpu.async_copy(h_hbm.at[sbuf[b]], rows[b], gsem[b]))
        for b in range(NBUF):
            gcps[b].wait()
            icps[b][1].wait()
            pltpu.sync_copy(rows[b], acc.at[dbuf[b]], add=True)

    plsc.subcore_barrier()

    r0 = s * ROWS_PER_TILE
    for sz in _pub_sizes(K):
        pltpu.sync_copy(acc.at[pl.ds(r0, sz)], rows[0].at[pl.ds(0, sz)])
        pltpu.sync_copy(rows[0].at[pl.ds(0, sz)], out_hbm.at[c, pl.ds(r0, sz)])
        r0 += sz


def _sc_deg_body(ept, dst_hbm, deg_hbm, dbuf, ones16, dsem, degacc):
    c = lax.axis_index("c")
    s = lax.axis_index("s")
    wid = c * NS + s
    ch_per_tile = ept // K

    zv = jnp.zeros((16,), jnp.float32)

    @pl.loop(0, K)
    def _(i):
        ones16[i, pl.ds(0, 16)] = zv

    r0 = s * ROWS_PER_TILE
    for sz in _pub_sizes(K):
        pltpu.sync_copy(ones16.at[pl.ds(0, sz)], degacc.at[pl.ds(r0, sz)])
        r0 += sz

    ov = jnp.ones((16,), jnp.float32)

    @pl.loop(0, K)
    def _(i):
        ones16[i, pl.ds(0, 16)] = ov

    plsc.subcore_barrier()

    @pl.loop(0, ch_per_tile)
    def _(ch):
        off = pl.multiple_of(wid * ept + ch * K, K)
        pltpu.async_copy(dst_hbm.at[pl.ds(off, K)], dbuf, dsem).wait()
        pltpu.sync_copy(ones16, degacc.at[dbuf], add=True)

    plsc.subcore_barrier()

    r0 = s * ROWS_PER_TILE
    for sz in _pub_sizes(K):
        pltpu.sync_copy(degacc.at[pl.ds(r0, sz)], ones16.at[pl.ds(0, sz)])
        pltpu.sync_copy(ones16.at[pl.ds(0, sz)], deg_hbm.at[c, pl.ds(r0, sz)])
        r0 += sz


def _make_sc_agg(ept):
    mesh = plsc.VectorSubcoreMesh(core_axis_name="c", subcore_axis_name="s")
    return pl.kernel(
        functools.partial(_sc_agg_body, ept),
        out_type=jax.ShapeDtypeStruct((NC, N_PAD, D), jnp.float32),
        mesh=mesh,
        scratch_types=(
            tuple(pltpu.VMEM((K,), jnp.int32) for _ in range(NBUF)),
            tuple(pltpu.VMEM((K,), jnp.int32) for _ in range(NBUF)),
            tuple(pltpu.VMEM((K, D), jnp.float32) for _ in range(NBUF)),
            tuple(pltpu.SemaphoreType.DMA for _ in range(NBUF)),
            tuple(pltpu.SemaphoreType.DMA for _ in range(NBUF)),
            tuple(pltpu.SemaphoreType.DMA for _ in range(NBUF)),
            pltpu.VMEM_SHARED((N_PAD, D), jnp.float32),
        ),
    )


def _make_sc_deg(ept):
    mesh = plsc.VectorSubcoreMesh(core_axis_name="c", subcore_axis_name="s")
    return pl.kernel(
        functools.partial(_sc_deg_body, ept),
        out_type=jax.ShapeDtypeStruct((NC, N_PAD, 16), jnp.float32),
        mesh=mesh,
        scratch_types=(
            pltpu.VMEM((K,), jnp.int32),
            pltpu.VMEM((K, 16), jnp.float32),
            pltpu.SemaphoreType.DMA,
            pltpu.VMEM_SHARED((N_PAD, 16), jnp.float32),
        ),
    )


def _tc_mean_layer_body(p0, p1, d0, d1, h, wl, bl, wr, out):
    deg = d0[0, :, 0:1] + d1[0, :, 0:1]
    inv = 1.0 / jnp.maximum(deg, 1.0)
    agg = (p0[0] + p1[0]) * inv
    y = (jnp.dot(agg, wl[...], preferred_element_type=jnp.float32)
         + bl[...]
         + jnp.dot(h[...], wr[...], preferred_element_type=jnp.float32))
    out[...] = jnp.maximum(y, 0.0)


def _tc_final_layer_body(p0, p1, h, wl, bl, wr, wlin, blin, out):
    agg = p0[0] + p1[0]
    y = (jnp.dot(agg, wl[...], preferred_element_type=jnp.float32)
         + bl[...]
         + jnp.dot(h[...], wr[...], preferred_element_type=jnp.float32))
    hh = jnp.maximum(y, 0.0)
    o = jnp.dot(hh, wlin[...], preferred_element_type=jnp.float32) + blin[...]
    e = jnp.exp(o - jnp.max(o, axis=1, keepdims=True))
    out[...] = e / jnp.sum(e, axis=1, keepdims=True)


_BM = 1264


def _part_spec(width, part):
    return pl.BlockSpec((1, _BM, width), lambda i, _p=part: (_p, i, 0))


def _row_spec(width):
    return pl.BlockSpec((_BM, width), lambda i: (i, 0))


def _full_spec(r, ccol):
    return pl.BlockSpec((r, ccol), lambda i: (0, 0))


def _tc_mean_layer(p, dp, h, wl, bl, wr):
    return pl.pallas_call(
        _tc_mean_layer_body,
        grid=(N_PAD // _BM,),
        in_specs=[
            _part_spec(D, 0), _part_spec(D, 1),
            _part_spec(16, 0), _part_spec(16, 1),
            _row_spec(D), _full_spec(D, D), _full_spec(1, D), _full_spec(D, D),
        ],
        out_specs=_row_spec(D),
        out_shape=jax.ShapeDtypeStruct((N_PAD, D), jnp.float32),
    )(p, p, dp, dp, h, wl, bl.reshape(1, D), wr)


def _tc_final_layer(p, h, wl, bl, wr, wlin, blin):
    return pl.pallas_call(
        _tc_final_layer_body,
        grid=(N_PAD // _BM,),
        in_specs=[
            _part_spec(D, 0), _part_spec(D, 1), _row_spec(D),
            _full_spec(D, D), _full_spec(1, D), _full_spec(D, D),
            _full_spec(D, 1), _full_spec(1, 1),
        ],
        out_specs=_row_spec(1),
        out_shape=jax.ShapeDtypeStruct((N_PAD, 1), jnp.float32),
    )(p, p, h, wl, bl.reshape(1, D), wr, wlin, blin.reshape(1, 1))


def kernel(x, edge_index, W1l, b1l, W1r, W2l, b2l, W2r, W3l, b3l, W3r,
           Wlin, blin):
    e = edge_index.shape[1]
    q = K * NBUF
    ept = -(-e // (NW * q)) * q
    e_pad = ept * NW
    src = edge_index[0].astype(jnp.int32)
    dst = edge_index[1].astype(jnp.int32)
    pad = e_pad - e
    if pad:
        fill = jnp.arange(pad, dtype=jnp.int32)
        src = jnp.concatenate([src, fill % N])
        dst = jnp.concatenate([dst, DUMMY_ROW + fill % (N_PAD - N)])
    xp = jnp.concatenate([x, jnp.zeros((N_PAD - N, D), x.dtype)])

    sc_agg = _make_sc_agg(ept)
    sc_deg = _make_sc_deg(ept)

    dp = sc_deg(dst)
    p = sc_agg(xp, src, dst)
    h1 = _tc_mean_layer(p, dp, xp, W1l, b1l, W1r)
    p = sc_agg(h1, src, dst)
    h2 = _tc_mean_layer(p, dp, h1, W2l, b2l, W2r)
    p = sc_agg(h2, src, dst)
    out = _tc_final_layer(p, h2, W3l, b3l, W3r, Wlin, blin)
    return out[:N]


# speedup vs baseline: 1.4871x; 1.0427x over previous
import functools

import jax
import jax.numpy as jnp
from jax import lax
from jax.experimental import pallas as pl
from jax.experimental.pallas import tpu as pltpu
from jax.experimental.pallas import tpu_sc as plsc

N = 10000
D = 128
NC = 2
NS = 16
NW = NC * NS
K = 128
NBUF = 3
ROWS_PER_TILE = 632
N_PAD = NS * ROWS_PER_TILE
DUMMY_ROW = N


def _pub_sizes(k):
    sizes = [k] * (ROWS_PER_TILE // k)
    if ROWS_PER_TILE % k:
        sizes.append(ROWS_PER_TILE % k)
    return sizes


def _sc_agg_body(ept, h_hbm, src_hbm, dst_hbm, out_hbm,
                 sbuf, dbuf, rows, isem, dsem, gsem, acc):
    c = lax.axis_index("c")
    s = lax.axis_index("s")
    wid = c * NS + s
    ch_per_tile = ept // K

    zv = jnp.zeros((16,), jnp.float32)

    @pl.loop(0, K)
    def _(i):
        for j in range(D // 16):
            rows[0][i, pl.ds(j * 16, 16)] = zv

    r0 = s * ROWS_PER_TILE
    for sz in _pub_sizes(K):
        pltpu.sync_copy(rows[0].at[pl.ds(0, sz)], acc.at[pl.ds(r0, sz)])
        r0 += sz

    plsc.subcore_barrier()

    @pl.loop(0, ch_per_tile, step=NBUF)
    def _(i):
        icps = []
        for b in range(NBUF):
            off = pl.multiple_of(wid * ept + (i + b) * K, K)
            icps.append((
                pltpu.async_copy(src_hbm.at[pl.ds(off, K)], sbuf[b], isem[b]),
                pltpu.async_copy(dst_hbm.at[pl.ds(off, K)], dbuf[b], dsem[b]),
            ))
        gcps = []
        for b in range(NBUF):
            icps[b][0].wait()
            gcps.append(pltpu.async_copy(h_hbm.at[sbuf[b]], rows[b], gsem[b]))
        for b in range(NBUF):
            gcps[b].wait()
            icps[b][1].wait()
            pltpu.sync_copy(rows[b], acc.at[dbuf[b]], add=True)

    plsc.subcore_barrier()

    r0 = s * ROWS_PER_TILE
    pltpu.sync_copy(acc.at[pl.ds(r0, ROWS_PER_TILE)],
                    out_hbm.at[c, pl.ds(r0, ROWS_PER_TILE)])


def _sc_deg_body(ept, dst_hbm, deg_hbm, dbuf, ones16, dsem, degacc):
    c = lax.axis_index("c")
    s = lax.axis_index("s")
    wid = c * NS + s
    ch_per_tile = ept // K

    zv = jnp.zeros((16,), jnp.float32)

    @pl.loop(0, K)
    def _(i):
        ones16[i, pl.ds(0, 16)] = zv

    r0 = s * ROWS_PER_TILE
    for sz in _pub_sizes(K):
        pltpu.sync_copy(ones16.at[pl.ds(0, sz)], degacc.at[pl.ds(r0, sz)])
        r0 += sz

    ov = jnp.ones((16,), jnp.float32)

    @pl.loop(0, K)
    def _(i):
        ones16[i, pl.ds(0, 16)] = ov

    plsc.subcore_barrier()

    @pl.loop(0, ch_per_tile, step=NBUF)
    def _(i):
        cps = []
        for b in range(NBUF):
            off = pl.multiple_of(wid * ept + (i + b) * K, K)
            cps.append(
                pltpu.async_copy(dst_hbm.at[pl.ds(off, K)], dbuf[b], dsem[b]))
        for b in range(NBUF):
            cps[b].wait()
            pltpu.sync_copy(ones16, degacc.at[dbuf[b]], add=True)

    plsc.subcore_barrier()

    r0 = s * ROWS_PER_TILE
    pltpu.sync_copy(degacc.at[pl.ds(r0, ROWS_PER_TILE)],
                    deg_hbm.at[c, pl.ds(r0, ROWS_PER_TILE)])


def _make_sc_agg(ept):
    mesh = plsc.VectorSubcoreMesh(core_axis_name="c", subcore_axis_name="s")
    return pl.kernel(
        functools.partial(_sc_agg_body, ept),
        out_type=jax.ShapeDtypeStruct((NC, N_PAD, D), jnp.float32),
        mesh=mesh,
        scratch_types=(
            tuple(pltpu.VMEM((K,), jnp.int32) for _ in range(NBUF)),
            tuple(pltpu.VMEM((K,), jnp.int32) for _ in range(NBUF)),
            tuple(pltpu.VMEM((K, D), jnp.float32) for _ in range(NBUF)),
            tuple(pltpu.SemaphoreType.DMA for _ in range(NBUF)),
            tuple(pltpu.SemaphoreType.DMA for _ in range(NBUF)),
            tuple(pltpu.SemaphoreType.DMA for _ in range(NBUF)),
            pltpu.VMEM_SHARED((N_PAD, D), jnp.float32),
        ),
    )


def _make_sc_deg(ept):
    mesh = plsc.VectorSubcoreMesh(core_axis_name="c", subcore_axis_name="s")
    return pl.kernel(
        functools.partial(_sc_deg_body, ept),
        out_type=jax.ShapeDtypeStruct((NC, N_PAD, 16), jnp.float32),
        mesh=mesh,
        scratch_types=(
            tuple(pltpu.VMEM((K,), jnp.int32) for _ in range(NBUF)),
            pltpu.VMEM((K, 16), jnp.float32),
            tuple(pltpu.SemaphoreType.DMA for _ in range(NBUF)),
            pltpu.VMEM_SHARED((N_PAD, 16), jnp.float32),
        ),
    )


def _tc_mean_layer_body(p0, p1, d0, d1, h, wl, bl, wr, out):
    deg = d0[0, :, 0:1] + d1[0, :, 0:1]
    inv = 1.0 / jnp.maximum(deg, 1.0)
    agg = (p0[0] + p1[0]) * inv
    y = (jnp.dot(agg, wl[...], preferred_element_type=jnp.float32)
         + bl[...]
         + jnp.dot(h[...], wr[...], preferred_element_type=jnp.float32))
    out[...] = jnp.maximum(y, 0.0)


def _tc_final_layer_body(p0, p1, h, wl, bl, wr, wlin, blin, out):
    agg = p0[0] + p1[0]
    y = (jnp.dot(agg, wl[...], preferred_element_type=jnp.float32)
         + bl[...]
         + jnp.dot(h[...], wr[...], preferred_element_type=jnp.float32))
    hh = jnp.maximum(y, 0.0)
    o = jnp.dot(hh, wlin[...], preferred_element_type=jnp.float32) + blin[...]
    e = jnp.exp(o - jnp.max(o, axis=1, keepdims=True))
    out[...] = e / jnp.sum(e, axis=1, keepdims=True)


_BM = 1264


def _part_spec(width, part):
    return pl.BlockSpec((1, _BM, width), lambda i, _p=part: (_p, i, 0))


def _row_spec(width):
    return pl.BlockSpec((_BM, width), lambda i: (i, 0))


def _full_spec(r, ccol):
    return pl.BlockSpec((r, ccol), lambda i: (0, 0))


def _tc_mean_layer(p, dp, h, wl, bl, wr):
    return pl.pallas_call(
        _tc_mean_layer_body,
        grid=(N_PAD // _BM,),
        in_specs=[
            _part_spec(D, 0), _part_spec(D, 1),
            _part_spec(16, 0), _part_spec(16, 1),
            _row_spec(D), _full_spec(D, D), _full_spec(1, D), _full_spec(D, D),
        ],
        out_specs=_row_spec(D),
        out_shape=jax.ShapeDtypeStruct((N_PAD, D), jnp.float32),
    )(p, p, dp, dp, h, wl, bl.reshape(1, D), wr)


def _tc_final_layer(p, h, wl, bl, wr, wlin, blin):
    return pl.pallas_call(
        _tc_final_layer_body,
        grid=(N_PAD // _BM,),
        in_specs=[
            _part_spec(D, 0), _part_spec(D, 1), _row_spec(D),
            _full_spec(D, D), _full_spec(1, D), _full_spec(D, D),
            _full_spec(D, 1), _full_spec(1, 1),
        ],
        out_specs=_row_spec(1),
        out_shape=jax.ShapeDtypeStruct((N_PAD, 1), jnp.float32),
    )(p, p, h, wl, bl.reshape(1, D), wr, wlin, blin.reshape(1, 1))


def kernel(x, edge_index, W1l, b1l, W1r, W2l, b2l, W2r, W3l, b3l, W3r,
           Wlin, blin):
    e = edge_index.shape[1]
    q = K * NBUF
    ept = -(-e // (NW * q)) * q
    e_pad = ept * NW
    src = edge_index[0].astype(jnp.int32)
    dst = edge_index[1].astype(jnp.int32)
    pad = e_pad - e
    if pad:
        fill = jnp.arange(pad, dtype=jnp.int32)
        src = jnp.concatenate([src, fill % N])
        dst = jnp.concatenate([dst, DUMMY_ROW + fill % (N_PAD - N)])
    xp = jnp.concatenate([x, jnp.zeros((N_PAD - N, D), x.dtype)])

    sc_agg = _make_sc_agg(ept)
    sc_deg = _make_sc_deg(ept)

    dp = sc_deg(dst)
    p = sc_agg(xp, src, dst)
    h1 = _tc_mean_layer(p, dp, xp, W1l, b1l, W1r)
    p = sc_agg(h1, src, dst)
    h2 = _tc_mean_layer(p, dp, h1, W2l, b2l, W2r)
    p = sc_agg(h2, src, dst)
    out = _tc_final_layer(p, h2, W3l, b3l, W3r, Wlin, blin)
    return out[:N]


# speedup vs baseline: 1.4880x; 1.0006x over previous
import functools

import jax
import jax.numpy as jnp
from jax import lax
from jax.experimental import pallas as pl
from jax.experimental.pallas import tpu as pltpu
from jax.experimental.pallas import tpu_sc as plsc

N = 10000
D = 128
NC = 2
NS = 16
NW = NC * NS
K = 192
NBUF = 2
ROWS_PER_TILE = 632
N_PAD = NS * ROWS_PER_TILE
DUMMY_ROW = N


def _pub_sizes(k):
    sizes = [k] * (ROWS_PER_TILE // k)
    if ROWS_PER_TILE % k:
        sizes.append(ROWS_PER_TILE % k)
    return sizes


def _sc_agg_body(ept, h_hbm, src_hbm, dst_hbm, out_hbm,
                 sbuf, dbuf, rows, isem, dsem, gsem, acc):
    c = lax.axis_index("c")
    s = lax.axis_index("s")
    wid = c * NS + s
    ch_per_tile = ept // K

    zv = jnp.zeros((16,), jnp.float32)

    @pl.loop(0, K)
    def _(i):
        for j in range(D // 16):
            rows[0][i, pl.ds(j * 16, 16)] = zv

    r0 = s * ROWS_PER_TILE
    for sz in _pub_sizes(K):
        pltpu.sync_copy(rows[0].at[pl.ds(0, sz)], acc.at[pl.ds(r0, sz)])
        r0 += sz

    plsc.subcore_barrier()

    @pl.loop(0, ch_per_tile, step=NBUF)
    def _(i):
        icps = []
        for b in range(NBUF):
            off = pl.multiple_of(wid * ept + (i + b) * K, K)
            icps.append((
                pltpu.async_copy(src_hbm.at[pl.ds(off, K)], sbuf[b], isem[b]),
                pltpu.async_copy(dst_hbm.at[pl.ds(off, K)], dbuf[b], dsem[b]),
            ))
        gcps = []
        for b in range(NBUF):
            icps[b][0].wait()
            gcps.append(pltpu.async_copy(h_hbm.at[sbuf[b]], rows[b], gsem[b]))
        for b in range(NBUF):
            gcps[b].wait()
            icps[b][1].wait()
            pltpu.sync_copy(rows[b], acc.at[dbuf[b]], add=True)

    plsc.subcore_barrier()

    r0 = s * ROWS_PER_TILE
    pltpu.sync_copy(acc.at[pl.ds(r0, ROWS_PER_TILE)],
                    out_hbm.at[c, pl.ds(r0, ROWS_PER_TILE)])


def _sc_deg_body(ept, dst_hbm, deg_hbm, dbuf, ones16, dsem, degacc):
    c = lax.axis_index("c")
    s = lax.axis_index("s")
    wid = c * NS + s
    ch_per_tile = ept // K

    zv = jnp.zeros((16,), jnp.float32)

    @pl.loop(0, K)
    def _(i):
        ones16[i, pl.ds(0, 16)] = zv

    r0 = s * ROWS_PER_TILE
    for sz in _pub_sizes(K):
        pltpu.sync_copy(ones16.at[pl.ds(0, sz)], degacc.at[pl.ds(r0, sz)])
        r0 += sz

    ov = jnp.ones((16,), jnp.float32)

    @pl.loop(0, K)
    def _(i):
        ones16[i, pl.ds(0, 16)] = ov

    plsc.subcore_barrier()

    @pl.loop(0, ch_per_tile, step=NBUF)
    def _(i):
        cps = []
        for b in range(NBUF):
            off = pl.multiple_of(wid * ept + (i + b) * K, K)
            cps.append(
                pltpu.async_copy(dst_hbm.at[pl.ds(off, K)], dbuf[b], dsem[b]))
        for b in range(NBUF):
            cps[b].wait()
            pltpu.sync_copy(ones16, degacc.at[dbuf[b]], add=True)

    plsc.subcore_barrier()

    r0 = s * ROWS_PER_TILE
    pltpu.sync_copy(degacc.at[pl.ds(r0, ROWS_PER_TILE)],
                    deg_hbm.at[c, pl.ds(r0, ROWS_PER_TILE)])


def _make_sc_agg(ept):
    mesh = plsc.VectorSubcoreMesh(core_axis_name="c", subcore_axis_name="s")
    return pl.kernel(
        functools.partial(_sc_agg_body, ept),
        out_type=jax.ShapeDtypeStruct((NC, N_PAD, D), jnp.float32),
        mesh=mesh,
        scratch_types=(
            tuple(pltpu.VMEM((K,), jnp.int32) for _ in range(NBUF)),
            tuple(pltpu.VMEM((K,), jnp.int32) for _ in range(NBUF)),
            tuple(pltpu.VMEM((K, D), jnp.float32) for _ in range(NBUF)),
            tuple(pltpu.SemaphoreType.DMA for _ in range(NBUF)),
            tuple(pltpu.SemaphoreType.DMA for _ in range(NBUF)),
            tuple(pltpu.SemaphoreType.DMA for _ in range(NBUF)),
            pltpu.VMEM_SHARED((N_PAD, D), jnp.float32),
        ),
    )


def _make_sc_deg(ept):
    mesh = plsc.VectorSubcoreMesh(core_axis_name="c", subcore_axis_name="s")
    return pl.kernel(
        functools.partial(_sc_deg_body, ept),
        out_type=jax.ShapeDtypeStruct((NC, N_PAD, 16), jnp.float32),
        mesh=mesh,
        scratch_types=(
            tuple(pltpu.VMEM((K,), jnp.int32) for _ in range(NBUF)),
            pltpu.VMEM((K, 16), jnp.float32),
            tuple(pltpu.SemaphoreType.DMA for _ in range(NBUF)),
            pltpu.VMEM_SHARED((N_PAD, 16), jnp.float32),
        ),
    )


def _tc_mean_layer_body(p0, p1, d0, d1, h, wl, bl, wr, out):
    deg = d0[0, :, 0:1] + d1[0, :, 0:1]
    inv = 1.0 / jnp.maximum(deg, 1.0)
    agg = (p0[0] + p1[0]) * inv
    y = (jnp.dot(agg, wl[...], preferred_element_type=jnp.float32)
         + bl[...]
         + jnp.dot(h[...], wr[...], preferred_element_type=jnp.float32))
    out[...] = jnp.maximum(y, 0.0)


def _tc_final_layer_body(p0, p1, h, wl, bl, wr, wlin, blin, out):
    agg = p0[0] + p1[0]
    y = (jnp.dot(agg, wl[...], preferred_element_type=jnp.float32)
         + bl[...]
         + jnp.dot(h[...], wr[...], preferred_element_type=jnp.float32))
    hh = jnp.maximum(y, 0.0)
    o = jnp.dot(hh, wlin[...], preferred_element_type=jnp.float32) + blin[...]
    e = jnp.exp(o - jnp.max(o, axis=1, keepdims=True))
    out[...] = e / jnp.sum(e, axis=1, keepdims=True)


_BM = 1264


def _part_spec(width, part):
    return pl.BlockSpec((1, _BM, width), lambda i, _p=part: (_p, i, 0))


def _row_spec(width):
    return pl.BlockSpec((_BM, width), lambda i: (i, 0))


def _full_spec(r, ccol):
    return pl.BlockSpec((r, ccol), lambda i: (0, 0))


def _tc_mean_layer(p, dp, h, wl, bl, wr):
    return pl.pallas_call(
        _tc_mean_layer_body,
        grid=(N_PAD // _BM,),
        in_specs=[
            _part_spec(D, 0), _part_spec(D, 1),
            _part_spec(16, 0), _part_spec(16, 1),
            _row_spec(D), _full_spec(D, D), _full_spec(1, D), _full_spec(D, D),
        ],
        out_specs=_row_spec(D),
        out_shape=jax.ShapeDtypeStruct((N_PAD, D), jnp.float32),
    )(p, p, dp, dp, h, wl, bl.reshape(1, D), wr)


def _tc_final_layer(p, h, wl, bl, wr, wlin, blin):
    return pl.pallas_call(
        _tc_final_layer_body,
        grid=(N_PAD // _BM,),
        in_specs=[
            _part_spec(D, 0), _part_spec(D, 1), _row_spec(D),
            _full_spec(D, D), _full_spec(1, D), _full_spec(D, D),
            _full_spec(D, 1), _full_spec(1, 1),
        ],
        out_specs=_row_spec(1),
        out_shape=jax.ShapeDtypeStruct((N_PAD, 1), jnp.float32),
    )(p, p, h, wl, bl.reshape(1, D), wr, wlin, blin.reshape(1, 1))


def kernel(x, edge_index, W1l, b1l, W1r, W2l, b2l, W2r, W3l, b3l, W3r,
           Wlin, blin):
    e = edge_index.shape[1]
    q = K * NBUF
    ept = -(-e // (NW * q)) * q
    e_pad = ept * NW
    src = edge_index[0].astype(jnp.int32)
    dst = edge_index[1].astype(jnp.int32)
    pad = e_pad - e
    if pad:
        fill = jnp.arange(pad, dtype=jnp.int32)
        src = jnp.concatenate([src, fill % N])
        dst = jnp.concatenate([dst, DUMMY_ROW + fill % (N_PAD - N)])
    xp = jnp.concatenate([x, jnp.zeros((N_PAD - N, D), x.dtype)])

    sc_agg = _make_sc_agg(ept)
    sc_deg = _make_sc_deg(ept)

    dp = sc_deg(dst)
    p = sc_agg(xp, src, dst)
    h1 = _tc_mean_layer(p, dp, xp, W1l, b1l, W1r)
    p = sc_agg(h1, src, dst)
    h2 = _tc_mean_layer(p, dp, h1, W2l, b2l, W2r)
    p = sc_agg(h2, src, dst)
    out = _tc_final_layer(p, h2, W3l, b3l, W3r, Wlin, blin)
    return out[:N]
